# baseline (speedup 1.0000x reference)
import functools
import math

import jax
import jax.numpy as jnp
from jax.experimental import pallas as pl
from jax.experimental.pallas import tpu as pltpu

_TOPK_FRAC = 0.1


def _body(mask_ref, h_ref, w1_ref, b1_ref, w2_ref, b2_ref,
          ge_ref, out_ref, adj_ref, adjt_ref, key_ref, s16_ref, mb_ref,
          *, n, k):
    ge = ge_ref[0]
    m = mask_ref[0, 0]
    h = h_ref[0]

    inp = jnp.concatenate(
        [ge, jnp.broadcast_to(h, (n, h.shape[1]))], axis=1) * m[:, None]
    e1 = jax.lax.dot_general(inp, w1_ref[...], (((1,), (0,)), ((), ())),
                             preferred_element_type=jnp.float32) + b1_ref[...]
    e2 = jax.lax.dot_general(inp, w2_ref[...], (((1,), (0,)), ((), ())),
                             preferred_element_type=jnp.float32) + b2_ref[...]

    adj_ref[...] = jax.lax.dot_general(e1, e2, (((1,), (1,)), ((), ())),
                                       preferred_element_type=jnp.float32)
    adjt_ref[...] = jax.lax.dot_general(e2, e1, (((1,), (1,)), ((), ())),
                                        preferred_element_type=jnp.float32)

    def key_to_f32(kk):
        return jax.lax.bitcast_convert_type(
            jnp.where(kk < 0, kk ^ jnp.int32(0x7FFFFFFF), kk), jnp.float32)

    ik = jax.lax.bitcast_convert_type(adjt_ref[...], jnp.int32)
    key0 = jnp.where(ik < 0, ik ^ jnp.int32(0x7FFFFFFF), ik)
    key_ref[...] = key0
    s16_ref[...] = jnp.right_shift(key0, 16).astype(jnp.int16)

    one_b = jnp.bfloat16(1.0)
    zero_b = jnp.bfloat16(0.0)
    ones8 = jnp.ones((8, n), jnp.bfloat16)

    def count16(cand):
        tb = cand.astype(jnp.int16)[None, :]
        mb_ref[...] = jnp.where(s16_ref[...] >= tb, one_b, zero_b)
        c = jax.lax.dot_general(ones8, mb_ref[...], (((1,), (0,)), ((), ())),
                                preferred_element_type=jnp.float32)
        return c[0]

    def select16(target):
        zero = jnp.zeros((n,), jnp.int32)
        p = jnp.where(count16(zero) >= target, zero,
                      jnp.full((n,), -32768, jnp.int32))

        def step(i, p):
            cand = p + (jnp.int32(1) << (jnp.int32(14) - i))
            return jnp.where(count16(cand) >= target, cand, p)

        return jax.lax.fori_loop(0, 15, step, p)

    p_hi = select16(jnp.full((n,), float(k), jnp.float32))
    c_hi = count16(p_hi + 1)

    key = key_ref[...]
    tied = jnp.right_shift(key, 16) == p_hi[None, :]
    rem = (key & jnp.int32(0xFFFF)) - jnp.int32(32768)
    s16_ref[...] = jnp.where(tied, rem, jnp.full_like(rem, -32768)
                             ).astype(jnp.int16)
    p_lo = select16(jnp.float32(k) - c_hi)

    pk = (p_hi << 16) | ((p_lo + jnp.int32(32768)) & jnp.int32(0xFFFF))
    thr = key_to_f32(pk)

    tcol = thr[None, :]
    s_acc = jnp.zeros((8, n), jnp.float32)
    c_acc = jnp.zeros((8, n), jnp.float32)
    m_acc = jnp.full((8, n), jnp.inf, jnp.float32)
    for j in range(n // 8):
        blk = adjt_ref[pl.ds(j * 8, 8), :]
        selb = blk >= tcol
        s_acc = s_acc + jnp.where(selb, blk, 0.0)
        c_acc = c_acc + jnp.where(selb, 1.0, 0.0)
        m_acc = jnp.minimum(m_acc, blk)
    s_raw = jnp.sum(s_acc, axis=0)
    cnt = jnp.sum(c_acc, axis=0)
    min_v = jnp.min(m_acc)

    rowsum = (s_raw - cnt * min_v) * m
    d = jnp.where(rowsum > 0, jax.lax.rsqrt(jnp.where(rowsum > 0, rowsum,
                                                      1.0)), 0.0)
    adj = adj_ref[...]
    sel = adj >= thr[:, None]
    out = jnp.where(sel, (adj - min_v) * (d[:, None] * d[None, :]), 0.0)
    out_ref[...] = out[None]


def kernel(graph_embed, mask, hidden_state, W1, b1, W2, b2):
    b, n, dg = graph_embed.shape
    dh = hidden_state.shape[1]
    dout = W1.shape[1]
    k = int(math.floor(n * _TOPK_FRAC))
    din = W1.shape[0]

    b1r = b1.reshape(1, dout)
    b2r = b2.reshape(1, dout)

    full2 = lambda i: (0, 0)
    batch3 = lambda i: (i, 0, 0)

    return pl.pallas_call(
        functools.partial(_body, n=n, k=k),
        grid=(b,),
        in_specs=[
            pl.BlockSpec((1, 1, n), batch3),
            pl.BlockSpec((1, 1, dh), batch3),
            pl.BlockSpec((din, dout), full2),
            pl.BlockSpec((1, dout), full2),
            pl.BlockSpec((din, dout), full2),
            pl.BlockSpec((1, dout), full2),
            pl.BlockSpec((1, n, dg), batch3),
        ],
        out_specs=pl.BlockSpec((1, n, n), batch3),
        out_shape=jax.ShapeDtypeStruct((b, n, n), jnp.float32),
        scratch_shapes=[
            pltpu.VMEM((n, n), jnp.float32),
            pltpu.VMEM((n, n), jnp.float32),
            pltpu.VMEM((n, n), jnp.int32),
            pltpu.VMEM((n, n), jnp.int16),
            pltpu.VMEM((n, n), jnp.bfloat16),
        ],
        compiler_params=pltpu.CompilerParams(
            dimension_semantics=("arbitrary",),
        ),
    )(mask.reshape(b, 1, n), hidden_state.reshape(b, 1, dh),
      W1, b1r, W2, b2r, graph_embed)


# speedup vs baseline: 1.3000x; 1.3000x over previous
import functools
import math

import jax
import jax.numpy as jnp
from jax.experimental import pallas as pl
from jax.experimental.pallas import tpu as pltpu

_TOPK_FRAC = 0.1


def _body(mask_ref, h_ref, w1_ref, b1_ref, w2_ref, b2_ref,
          ge_ref, out_ref, adj_ref, adjt_ref, key_ref, s16_ref, *, n, k):
    ge = ge_ref[0]
    m = mask_ref[0, 0]
    h = h_ref[0]

    inp = jnp.concatenate(
        [ge, jnp.broadcast_to(h, (n, h.shape[1]))], axis=1) * m[:, None]
    e1 = jax.lax.dot_general(inp, w1_ref[...], (((1,), (0,)), ((), ())),
                             preferred_element_type=jnp.float32) + b1_ref[...]
    e2 = jax.lax.dot_general(inp, w2_ref[...], (((1,), (0,)), ((), ())),
                             preferred_element_type=jnp.float32) + b2_ref[...]

    adj_ref[...] = jax.lax.dot_general(e1, e2, (((1,), (1,)), ((), ())),
                                       preferred_element_type=jnp.float32)
    adjt_ref[...] = jax.lax.dot_general(e2, e1, (((1,), (1,)), ((), ())),
                                        preferred_element_type=jnp.float32)

    def key_to_f32(kk):
        return jax.lax.bitcast_convert_type(
            jnp.where(kk < 0, kk ^ jnp.int32(0x7FFFFFFF), kk), jnp.float32)

    ik = jax.lax.bitcast_convert_type(adjt_ref[...], jnp.int32)
    key0 = jnp.where(ik < 0, ik ^ jnp.int32(0x7FFFFFFF), ik)
    key_ref[...] = key0
    s16_ref[...] = jnp.right_shift(key0, 16).astype(jnp.int16)
    min_key = jnp.min(key0)

    one_b = jnp.bfloat16(1.0)
    zero_b = jnp.bfloat16(0.0)
    ones_col = jnp.ones((n, 8), jnp.float32)

    def count16(cand):
        tb = cand.astype(jnp.int16)[None, :]
        acc0 = jnp.zeros((32, n), jnp.bfloat16)
        acc1 = jnp.zeros((32, n), jnp.bfloat16)
        for j in range(n // 64):
            blk = s16_ref[pl.ds(j * 64, 64), :]
            acc0 = acc0 + jnp.where(blk[:32] >= tb, one_b, zero_b)
            acc1 = acc1 + jnp.where(blk[32:] >= tb, one_b, zero_b)
        acc = (acc0 + acc1).astype(jnp.float32)
        return jnp.sum(acc, axis=0)

    def select16(target):
        zero = jnp.zeros((n,), jnp.int32)
        p = jnp.where(count16(zero) >= target, zero,
                      jnp.full((n,), -32768, jnp.int32))

        def step(i, p):
            cand = p + (jnp.int32(1) << (jnp.int32(14) - i))
            return jnp.where(count16(cand) >= target, cand, p)

        return jax.lax.fori_loop(0, 15, step, p)

    p_hi = select16(jnp.full((n,), float(k), jnp.float32))
    c_hi = count16(p_hi + 1)

    key = key_ref[...]
    tied = jnp.right_shift(key, 16) == p_hi[None, :]
    rem = (key & jnp.int32(0xFFFF)) - jnp.int32(32768)
    s16_ref[...] = jnp.where(tied, rem, jnp.full_like(rem, -32768)
                             ).astype(jnp.int16)
    p_lo = select16(jnp.float32(k) - c_hi)

    pk = (p_hi << 16) | ((p_lo + jnp.int32(32768)) & jnp.int32(0xFFFF))
    thr = key_to_f32(pk)
    min_v = key_to_f32(min_key)

    adj = adj_ref[...]
    p_mat = jnp.where(adj >= thr[:, None], adj - min_v, 0.0)
    adjt_ref[...] = p_mat
    rs = jax.lax.dot_general(p_mat, ones_col, (((1,), (0,)), ((), ())),
                             preferred_element_type=jnp.float32)
    rowsum = rs[:, 0] * m
    d = jnp.where(rowsum > 0, jax.lax.rsqrt(jnp.where(rowsum > 0, rowsum,
                                                      1.0)), 0.0)
    out = (adjt_ref[...] * d[None, :]) * d[:, None]
    out_ref[...] = out[None]


def kernel(graph_embed, mask, hidden_state, W1, b1, W2, b2):
    b, n, dg = graph_embed.shape
    dh = hidden_state.shape[1]
    dout = W1.shape[1]
    k = int(math.floor(n * _TOPK_FRAC))
    din = W1.shape[0]

    b1r = b1.reshape(1, dout)
    b2r = b2.reshape(1, dout)

    full2 = lambda i: (0, 0)
    batch3 = lambda i: (i, 0, 0)

    return pl.pallas_call(
        functools.partial(_body, n=n, k=k),
        grid=(b,),
        in_specs=[
            pl.BlockSpec((1, 1, n), batch3),
            pl.BlockSpec((1, 1, dh), batch3),
            pl.BlockSpec((din, dout), full2),
            pl.BlockSpec((1, dout), full2),
            pl.BlockSpec((din, dout), full2),
            pl.BlockSpec((1, dout), full2),
            pl.BlockSpec((1, n, dg), batch3),
        ],
        out_specs=pl.BlockSpec((1, n, n), batch3),
        out_shape=jax.ShapeDtypeStruct((b, n, n), jnp.float32),
        scratch_shapes=[
            pltpu.VMEM((n, n), jnp.float32),
            pltpu.VMEM((n, n), jnp.float32),
            pltpu.VMEM((n, n), jnp.int32),
            pltpu.VMEM((n, n), jnp.int16),
        ],
        compiler_params=pltpu.CompilerParams(
            dimension_semantics=("arbitrary",),
        ),
    )(mask.reshape(b, 1, n), hidden_state.reshape(b, 1, dh),
      W1, b1r, W2, b2r, graph_embed)


# speedup vs baseline: 1.4279x; 1.0983x over previous
import functools
import math

import jax
import jax.numpy as jnp
from jax.experimental import pallas as pl
from jax.experimental.pallas import tpu as pltpu

_TOPK_FRAC = 0.1


def _body(mask_ref, h_ref, w1_ref, b1_ref, w2_ref, b2_ref,
          ge_ref, out_ref, adj_ref, adjt_ref, key_ref, s16_ref, *, n, k):
    ge = ge_ref[0]
    m = mask_ref[0, 0]
    h = h_ref[0]

    inp = jnp.concatenate(
        [ge, jnp.broadcast_to(h, (n, h.shape[1]))], axis=1) * m[:, None]
    e1 = jax.lax.dot_general(inp, w1_ref[...], (((1,), (0,)), ((), ())),
                             preferred_element_type=jnp.float32) + b1_ref[...]
    e2 = jax.lax.dot_general(inp, w2_ref[...], (((1,), (0,)), ((), ())),
                             preferred_element_type=jnp.float32) + b2_ref[...]

    adj_ref[...] = jax.lax.dot_general(e1, e2, (((1,), (1,)), ((), ())),
                                       preferred_element_type=jnp.float32)
    adjt_ref[...] = jax.lax.dot_general(e2, e1, (((1,), (1,)), ((), ())),
                                        preferred_element_type=jnp.float32)

    def key_to_f32(kk):
        return jax.lax.bitcast_convert_type(
            jnp.where(kk < 0, kk ^ jnp.int32(0x7FFFFFFF), kk), jnp.float32)

    ik = jax.lax.bitcast_convert_type(adjt_ref[...], jnp.int32)
    key0 = jnp.where(ik < 0, ik ^ jnp.int32(0x7FFFFFFF), ik)
    key_ref[...] = key0
    s16_ref[...] = jnp.right_shift(key0, 16).astype(jnp.int16)
    min_key = jnp.min(key0)

    one_b = jnp.bfloat16(1.0)
    zero_b = jnp.bfloat16(0.0)

    def count16(cand):
        tb = cand.astype(jnp.int16)[None, :]
        acc0 = jnp.zeros((32, n), jnp.bfloat16)
        acc1 = jnp.zeros((32, n), jnp.bfloat16)
        for j in range(n // 64):
            blk = s16_ref[pl.ds(j * 64, 64), :]
            acc0 = acc0 + jnp.where(blk[:32] >= tb, one_b, zero_b)
            acc1 = acc1 + jnp.where(blk[32:] >= tb, one_b, zero_b)
        acc = (acc0 + acc1).astype(jnp.float32)
        return jnp.sum(acc, axis=0)

    def select16(target):
        zero = jnp.zeros((n,), jnp.int32)
        p = jnp.where(count16(zero) >= target, zero,
                      jnp.full((n,), -32768, jnp.int32))

        def step(i, p):
            cand = p + (jnp.int32(1) << (jnp.int32(14) - i))
            return jnp.where(count16(cand) >= target, cand, p)

        return jax.lax.fori_loop(0, 15, step, p)

    p_hi = select16(jnp.full((n,), float(k), jnp.float32))
    c_hi = count16(p_hi + 1)

    key = key_ref[...]
    tied = jnp.right_shift(key, 16) == p_hi[None, :]
    rem = (key & jnp.int32(0xFFFF)) - jnp.int32(32768)
    s16_ref[...] = jnp.where(tied, rem, jnp.full_like(rem, -32768)
                             ).astype(jnp.int16)
    p_lo = select16(jnp.float32(k) - c_hi)

    pk = (p_hi << 16) | ((p_lo + jnp.int32(32768)) & jnp.int32(0xFFFF))
    thr = key_to_f32(pk)
    min_v = key_to_f32(min_key)

    tcol = thr[None, :]
    s_acc = jnp.zeros((8, n), jnp.float32)
    for j in range(n // 8):
        blk = adjt_ref[pl.ds(j * 8, 8), :]
        s_acc = s_acc + jnp.where(blk >= tcol, blk - min_v, 0.0)
    rowsum = jnp.sum(s_acc, axis=0) * m
    d = jnp.where(rowsum > 0, jax.lax.rsqrt(jnp.where(rowsum > 0, rowsum,
                                                      1.0)), 0.0)
    adj = adj_ref[...]
    sel = adj >= thr[:, None]
    out = jnp.where(sel, (adj - min_v) * (d[:, None] * d[None, :]), 0.0)
    out_ref[...] = out[None]


def kernel(graph_embed, mask, hidden_state, W1, b1, W2, b2):
    b, n, dg = graph_embed.shape
    dh = hidden_state.shape[1]
    dout = W1.shape[1]
    k = int(math.floor(n * _TOPK_FRAC))
    din = W1.shape[0]

    b1r = b1.reshape(1, dout)
    b2r = b2.reshape(1, dout)

    full2 = lambda i: (0, 0)
    batch3 = lambda i: (i, 0, 0)

    return pl.pallas_call(
        functools.partial(_body, n=n, k=k),
        grid=(b,),
        in_specs=[
            pl.BlockSpec((1, 1, n), batch3),
            pl.BlockSpec((1, 1, dh), batch3),
            pl.BlockSpec((din, dout), full2),
            pl.BlockSpec((1, dout), full2),
            pl.BlockSpec((din, dout), full2),
            pl.BlockSpec((1, dout), full2),
            pl.BlockSpec((1, n, dg), batch3),
        ],
        out_specs=pl.BlockSpec((1, n, n), batch3),
        out_shape=jax.ShapeDtypeStruct((b, n, n), jnp.float32),
        scratch_shapes=[
            pltpu.VMEM((n, n), jnp.float32),
            pltpu.VMEM((n, n), jnp.float32),
            pltpu.VMEM((n, n), jnp.int32),
            pltpu.VMEM((n, n), jnp.int16),
        ],
        compiler_params=pltpu.CompilerParams(
            dimension_semantics=("arbitrary",),
        ),
    )(mask.reshape(b, 1, n), hidden_state.reshape(b, 1, dh),
      W1, b1r, W2, b2r, graph_embed)
